# baseline (speedup 1.0000x reference)
import functools

import jax
import jax.numpy as jnp
from jax import lax
from jax.experimental import pallas as pl
from jax.experimental.pallas import tpu as pltpu
from jax.experimental.pallas import tpu_sc as plsc

N_NODES = 10000
N_EDGES = 320000
D_FEAT = 128
D_HIDDEN = 128
N_CLASSES = 16

NUM_CORES = 2
NUM_SUBCORES = 16
NW = NUM_CORES * NUM_SUBCORES
EDGES_PER_W = N_EDGES // NW
CHUNK = 125
NCHUNKS = EDGES_PER_W // CHUNK
N_PAD = 10240
ROWS_PER_TILE = N_PAD // NUM_SUBCORES
ZROWS = 128

_MM_BLOCK_M = 2000


def _matmul_body(x_ref, w_ref, o_ref):
    o_ref[...] = lax.dot_general(
        x_ref[...], w_ref[...], (((1,), (0,)), ((), ())),
        precision=lax.Precision.HIGHEST, preferred_element_type=jnp.float32)


def _matmul(x, w):
    m, k = x.shape
    n = w.shape[1]
    return pl.pallas_call(
        _matmul_body,
        grid=(m // _MM_BLOCK_M,),
        in_specs=[
            pl.BlockSpec((_MM_BLOCK_M, k), lambda i: (i, 0)),
            pl.BlockSpec((k, n), lambda i: (0, 0)),
        ],
        out_specs=pl.BlockSpec((_MM_BLOCK_M, n), lambda i: (i, 0)),
        out_shape=jax.ShapeDtypeStruct((m, n), jnp.float32),
    )(x, w)


_MID_BLOCK_M = 2048


def _fused_mid_body(p0_ref, p1_ref, b_ref, m_ref, w_ref, o_ref):
    h = jnp.maximum(p0_ref[...] + p1_ref[...] + b_ref[...], 0.0) * m_ref[...]
    o_ref[...] = lax.dot_general(
        h, w_ref[...], (((1,), (0,)), ((), ())),
        precision=lax.Precision.HIGHEST, preferred_element_type=jnp.float32)


def _fused_mid(p0, p1, b1, mult, w2pad):
    m, k = p0.shape
    n = w2pad.shape[1]
    return pl.pallas_call(
        _fused_mid_body,
        grid=(m // _MID_BLOCK_M,),
        in_specs=[
            pl.BlockSpec((_MID_BLOCK_M, k), lambda i: (i, 0)),
            pl.BlockSpec((_MID_BLOCK_M, k), lambda i: (i, 0)),
            pl.BlockSpec((1, k), lambda i: (0, 0)),
            pl.BlockSpec((_MID_BLOCK_M, k), lambda i: (i, 0)),
            pl.BlockSpec((k, n), lambda i: (0, 0)),
        ],
        out_specs=pl.BlockSpec((_MID_BLOCK_M, n), lambda i: (i, 0)),
        out_shape=jax.ShapeDtypeStruct((m, n), jnp.float32),
    )(p0, p1, b1, mult, w2pad)


def _final_body(q0_ref, q1_ref, b_ref, o_ref):
    o_ref[...] = q0_ref[...] + q1_ref[...] + b_ref[...]


def _final_sum(q0, q1, b2t):
    return pl.pallas_call(
        _final_body,
        out_shape=jax.ShapeDtypeStruct(q0.shape, jnp.float32),
    )(q0, q1, b2t)


C128 = 100
NC128 = EDGES_PER_W // C128
NPHASES = 5
PC128 = NC128 // NPHASES
PIPE_ITERS = (PC128 - 2) // 3
assert PC128 == 3 * PIPE_ITERS + 2


def _make_spmm(d):
    mesh = plsc.VectorSubcoreMesh(core_axis_name="c", subcore_axis_name="s")
    scratch = [
        pltpu.VMEM((PC128, C128), jnp.int32),
        pltpu.VMEM((PC128, C128), jnp.int32),
        pltpu.VMEM((C128, d), jnp.float32),
        pltpu.VMEM((C128, d), jnp.float32),
        pltpu.VMEM((C128, d), jnp.float32),
        pltpu.VMEM_SHARED((N_PAD, d), jnp.float32),
        pltpu.SemaphoreType.DMA,
        pltpu.SemaphoreType.DMA,
        pltpu.SemaphoreType.DMA,
    ]

    @functools.partial(
        pl.kernel,
        out_type=jax.ShapeDtypeStruct(
            (NUM_CORES, NUM_SUBCORES, ROWS_PER_TILE, d), jnp.float32),
        mesh=mesh,
        scratch_types=scratch,
    )
    def spmm(src_hbm, dst_hbm, sup_hbm, out_hbm,
             src_v, dst_v, buf0, buf1, buf2, acc_s, sem0, sem1, sem2):
        core = lax.axis_index("c")
        sub = lax.axis_index("s")
        wid = core * NUM_SUBCORES + sub
        stripe = pl.ds(sub * ROWS_PER_TILE, ROWS_PER_TILE)

        @pl.loop(0, C128)
        def _zrow(i):
            @pl.loop(0, d, step=16)
            def _zcol(j):
                buf0[i, pl.ds(j, 16)] = jnp.zeros((16,), jnp.float32)

        @pl.loop(0, 6)
        def _zcopy(k):
            pltpu.sync_copy(
                buf0, acc_s.at[pl.ds(sub * ROWS_PER_TILE + k * C128, C128)])

        pltpu.sync_copy(
            buf0.at[pl.ds(0, 40)],
            acc_s.at[pl.ds(sub * ROWS_PER_TILE + 6 * C128, 40)])

        plsc.subcore_barrier()

        bufs = (buf0, buf1, buf2)
        sems = (sem0, sem1, sem2)

        def gather(j, b):
            pltpu.async_copy(sup_hbm.at[src_v.at[j]], bufs[b], sems[b])

        def consume(j, b):
            pltpu.make_async_copy(
                sup_hbm.at[src_v.at[j]], bufs[b], sems[b]).wait()
            pltpu.sync_copy(bufs[b], acc_s.at[dst_v.at[j]], add=True)

        for phase in range(NPHASES):
            pltpu.sync_copy(src_hbm.at[wid, phase], src_v)
            pltpu.sync_copy(dst_hbm.at[wid, phase], dst_v)

            gather(0, 0)
            gather(1, 1)

            @pl.loop(0, PIPE_ITERS)
            def _trip(k):
                j = 3 * k
                gather(j + 2, 2)
                consume(j, 0)
                gather(j + 3, 0)
                consume(j + 1, 1)
                gather(j + 4, 1)
                consume(j + 2, 2)

            consume(PC128 - 2, 0)
            consume(PC128 - 1, 1)

        plsc.subcore_barrier()

        pltpu.sync_copy(acc_s.at[stripe], out_hbm.at[core, sub])

    return spmm


def _make_spmm_narrow(d):
    mesh = plsc.VectorSubcoreMesh(core_axis_name="c", subcore_axis_name="s")

    @functools.partial(
        pl.kernel,
        out_type=jax.ShapeDtypeStruct(
            (NUM_CORES, NUM_SUBCORES, ROWS_PER_TILE, d), jnp.float32),
        mesh=mesh,
        scratch_types=[
            pltpu.VMEM((NCHUNKS, CHUNK), jnp.int32),
            pltpu.VMEM((NCHUNKS, CHUNK), jnp.int32),
            pltpu.VMEM((ZROWS, d), jnp.float32),
            pltpu.VMEM((ZROWS, d), jnp.float32),
            pltpu.VMEM_SHARED((N_PAD, d), jnp.float32),
            pltpu.SemaphoreType.DMA,
            pltpu.SemaphoreType.DMA,
        ],
        compiler_params=pltpu.CompilerParams(use_tc_tiling_on_sc=False),
    )
    def spmm(src_hbm, dst_hbm, sup_hbm, out_hbm,
             src_v, dst_v, buf0, buf1, acc_s, sem0, sem1):
        core = lax.axis_index("c")
        sub = lax.axis_index("s")
        wid = core * NUM_SUBCORES + sub
        stripe = pl.ds(sub * ROWS_PER_TILE, ROWS_PER_TILE)

        for buf in (buf0, buf1):
            @pl.loop(0, ZROWS)
            def _zrow(i, buf=buf):
                @pl.loop(0, d, step=16)
                def _zcol(j, buf=buf):
                    buf[i, pl.ds(j, 16)] = jnp.zeros((16,), jnp.float32)

        @pl.loop(0, ROWS_PER_TILE // ZROWS)
        def _zcopy(k):
            pltpu.sync_copy(
                buf0, acc_s.at[pl.ds(sub * ROWS_PER_TILE + k * ZROWS, ZROWS)])

        pltpu.sync_copy(src_hbm.at[wid], src_v)
        pltpu.sync_copy(dst_hbm.at[wid], dst_v)
        plsc.subcore_barrier()

        cbuf = pl.ds(0, CHUNK)

        def gather(idx_row, buf, sem):
            pltpu.async_copy(sup_hbm.at[idx_row], buf.at[cbuf], sem)

        def gwait(idx_row, buf, sem):
            pltpu.make_async_copy(sup_hbm.at[idx_row], buf.at[cbuf], sem).wait()

        def scatter(buf, idx_row):
            pltpu.sync_copy(buf.at[cbuf], acc_s.at[idx_row], add=True)

        gather(src_v.at[0], buf0, sem0)

        @pl.loop(0, NCHUNKS // 2 - 1)
        def _pair(k):
            j = 2 * k
            gather(src_v.at[j + 1], buf1, sem1)
            gwait(src_v.at[j], buf0, sem0)
            scatter(buf0, dst_v.at[j])
            gather(src_v.at[j + 2], buf0, sem0)
            gwait(src_v.at[j + 1], buf1, sem1)
            scatter(buf1, dst_v.at[j + 1])

        jt = NCHUNKS - 2
        gather(src_v.at[jt + 1], buf1, sem1)
        gwait(src_v.at[jt], buf0, sem0)
        scatter(buf0, dst_v.at[jt])
        gwait(src_v.at[jt + 1], buf1, sem1)
        scatter(buf1, dst_v.at[jt + 1])

        plsc.subcore_barrier()
        pltpu.sync_copy(acc_s.at[stripe], out_hbm.at[core, sub])

    return spmm


_spmm128 = _make_spmm(D_HIDDEN)
_spmm16 = _make_spmm_narrow(N_CLASSES)


def kernel(x, edge_index, W1, b1, W2, b2):
    src128 = edge_index[0].reshape(NW, NPHASES, PC128, C128)
    dst128 = edge_index[1].reshape(NW, NPHASES, PC128, C128)
    src = edge_index[0].reshape(NW, NCHUNKS, CHUNK)
    dst = edge_index[1].reshape(NW, NCHUNKS, CHUNK)

    support1 = _matmul(x, W1)

    p1 = _spmm128(src128, dst128, support1).reshape(NUM_CORES, N_PAD, D_HIDDEN)

    mask = jax.random.bernoulli(jax.random.key(42), 0.5, (N_NODES, D_HIDDEN))
    mult = jnp.pad(mask.astype(jnp.float32) * 2.0,
                   ((0, N_PAD - N_NODES), (0, 0)))
    w2pad = jnp.zeros((D_HIDDEN, D_HIDDEN), jnp.float32).at[:, :N_CLASSES].set(W2)
    support2 = _fused_mid(p1[0], p1[1], b1.reshape(1, -1), mult, w2pad)
    support2 = support2[:, :N_CLASSES]

    p2 = _spmm16(src, dst, support2).reshape(NUM_CORES, N_PAD, N_CLASSES)

    q0 = p2[0, :N_NODES].reshape(-1, 128)
    q1 = p2[1, :N_NODES].reshape(-1, 128)
    b2t = jnp.tile(b2, 128 // N_CLASSES).reshape(1, 128)
    out = _final_sum(q0, q1, jnp.broadcast_to(b2t, q0.shape))
    return out.reshape(N_NODES, N_CLASSES)


# speedup vs baseline: 1.0480x; 1.0480x over previous
import functools

import jax
import jax.numpy as jnp
from jax import lax
from jax.experimental import pallas as pl
from jax.experimental.pallas import tpu as pltpu
from jax.experimental.pallas import tpu_sc as plsc

N_NODES = 10000
N_EDGES = 320000
D_FEAT = 128
D_HIDDEN = 128
N_CLASSES = 16

NUM_CORES = 2
NUM_SUBCORES = 16
NW = NUM_CORES * NUM_SUBCORES
EDGES_PER_W = N_EDGES // NW
CHUNK = 125
NCHUNKS = EDGES_PER_W // CHUNK
N_PAD = 10240
ROWS_PER_TILE = N_PAD // NUM_SUBCORES
ZROWS = 128

_MM_BLOCK_M = 2000


def _matmul_body(x_ref, w_ref, o_ref):
    o_ref[...] = lax.dot_general(
        x_ref[...], w_ref[...], (((1,), (0,)), ((), ())),
        precision=lax.Precision.HIGHEST, preferred_element_type=jnp.float32)


def _matmul(x, w):
    m, k = x.shape
    n = w.shape[1]
    return pl.pallas_call(
        _matmul_body,
        grid=(m // _MM_BLOCK_M,),
        in_specs=[
            pl.BlockSpec((_MM_BLOCK_M, k), lambda i: (i, 0)),
            pl.BlockSpec((k, n), lambda i: (0, 0)),
        ],
        out_specs=pl.BlockSpec((_MM_BLOCK_M, n), lambda i: (i, 0)),
        out_shape=jax.ShapeDtypeStruct((m, n), jnp.float32),
    )(x, w)


_MID_BLOCK_M = 2048


def _fused_mid_body(p0_ref, p1_ref, b_ref, m_ref, w_ref, o_ref):
    h = jnp.maximum(p0_ref[...] + p1_ref[...] + b_ref[...], 0.0) * m_ref[...]
    o_ref[...] = lax.dot_general(
        h, w_ref[...], (((1,), (0,)), ((), ())),
        precision=lax.Precision.HIGHEST, preferred_element_type=jnp.float32)


def _fused_mid(p0, p1, b1, mult, w2pad):
    m, k = p0.shape
    n = w2pad.shape[1]
    return pl.pallas_call(
        _fused_mid_body,
        grid=(m // _MID_BLOCK_M,),
        in_specs=[
            pl.BlockSpec((_MID_BLOCK_M, k), lambda i: (i, 0)),
            pl.BlockSpec((_MID_BLOCK_M, k), lambda i: (i, 0)),
            pl.BlockSpec((1, k), lambda i: (0, 0)),
            pl.BlockSpec((_MID_BLOCK_M, k), lambda i: (i, 0)),
            pl.BlockSpec((k, n), lambda i: (0, 0)),
        ],
        out_specs=pl.BlockSpec((_MID_BLOCK_M, n), lambda i: (i, 0)),
        out_shape=jax.ShapeDtypeStruct((m, n), jnp.float32),
    )(p0, p1, b1, mult, w2pad)


def _final_body(q0_ref, q1_ref, b_ref, o_ref):
    o_ref[...] = q0_ref[...] + q1_ref[...] + b_ref[...]


def _final_sum(q0, q1, b2t):
    return pl.pallas_call(
        _final_body,
        out_shape=jax.ShapeDtypeStruct(q0.shape, jnp.float32),
    )(q0, q1, b2t)


C128 = 100
NC128 = EDGES_PER_W // C128
NPHASES = 5
PC128 = NC128 // NPHASES
PIPE_ITERS = (PC128 - 2) // 3
assert PC128 == 3 * PIPE_ITERS + 2


def _make_spmm(d):
    mesh = plsc.VectorSubcoreMesh(core_axis_name="c", subcore_axis_name="s")
    scratch = [
        pltpu.VMEM((PC128, C128), jnp.int32),
        pltpu.VMEM((PC128, C128), jnp.int32),
        pltpu.VMEM((C128, d), jnp.float32),
        pltpu.VMEM((C128, d), jnp.float32),
        pltpu.VMEM((C128, d), jnp.float32),
        pltpu.VMEM_SHARED((N_PAD, d), jnp.float32),
        pltpu.SemaphoreType.DMA,
        pltpu.SemaphoreType.DMA,
        pltpu.SemaphoreType.DMA,
    ]

    @functools.partial(
        pl.kernel,
        out_type=jax.ShapeDtypeStruct(
            (NUM_CORES, NUM_SUBCORES, ROWS_PER_TILE, d), jnp.float32),
        mesh=mesh,
        scratch_types=scratch,
    )
    def spmm(src_hbm, dst_hbm, sup_hbm, out_hbm,
             src_v, dst_v, buf0, buf1, buf2, acc_s, sem0, sem1, sem2):
        core = lax.axis_index("c")
        sub = lax.axis_index("s")
        wid = core * NUM_SUBCORES + sub
        stripe = pl.ds(sub * ROWS_PER_TILE, ROWS_PER_TILE)

        @pl.loop(0, C128)
        def _zrow(i):
            @pl.loop(0, d, step=16)
            def _zcol(j):
                buf0[i, pl.ds(j, 16)] = jnp.zeros((16,), jnp.float32)

        @pl.loop(0, 6)
        def _zcopy(k):
            pltpu.sync_copy(
                buf0, acc_s.at[pl.ds(sub * ROWS_PER_TILE + k * C128, C128)])

        pltpu.sync_copy(
            buf0.at[pl.ds(0, 40)],
            acc_s.at[pl.ds(sub * ROWS_PER_TILE + 6 * C128, 40)])

        plsc.subcore_barrier()

        bufs = (buf0, buf1, buf2)
        sems = (sem0, sem1, sem2)

        def gather(j, b):
            pltpu.async_copy(sup_hbm.at[src_v.at[j]], bufs[b], sems[b])

        def consume(j, b):
            pltpu.make_async_copy(
                sup_hbm.at[src_v.at[j]], bufs[b], sems[b]).wait()
            pltpu.sync_copy(bufs[b], acc_s.at[dst_v.at[j]], add=True)

        for phase in range(NPHASES):
            pltpu.sync_copy(src_hbm.at[wid, phase], src_v)
            pltpu.sync_copy(dst_hbm.at[wid, phase], dst_v)

            gather(0, 0)
            gather(1, 1)

            @pl.loop(0, PIPE_ITERS)
            def _trip(k):
                j = 3 * k
                gather(j + 2, 2)
                consume(j, 0)
                gather(j + 3, 0)
                consume(j + 1, 1)
                gather(j + 4, 1)
                consume(j + 2, 2)

            consume(PC128 - 2, 0)
            consume(PC128 - 1, 1)

        plsc.subcore_barrier()

        pltpu.sync_copy(acc_s.at[stripe], out_hbm.at[core, sub])

    return spmm


def _make_spmm_narrow(d):
    mesh = plsc.VectorSubcoreMesh(core_axis_name="c", subcore_axis_name="s")

    @functools.partial(
        pl.kernel,
        out_type=jax.ShapeDtypeStruct(
            (NUM_CORES, NUM_SUBCORES, ROWS_PER_TILE, d), jnp.float32),
        mesh=mesh,
        scratch_types=[
            pltpu.VMEM((NCHUNKS, CHUNK), jnp.int32),
            pltpu.VMEM((NCHUNKS, CHUNK), jnp.int32),
            pltpu.VMEM((ZROWS, d), jnp.float32),
            pltpu.VMEM((ZROWS, d), jnp.float32),
            pltpu.VMEM((ZROWS, d), jnp.float32),
            pltpu.VMEM_SHARED((N_PAD, d), jnp.float32),
            pltpu.SemaphoreType.DMA,
            pltpu.SemaphoreType.DMA,
            pltpu.SemaphoreType.DMA,
        ],
        compiler_params=pltpu.CompilerParams(use_tc_tiling_on_sc=False),
    )
    def spmm(src_hbm, dst_hbm, sup_hbm, out_hbm,
             src_v, dst_v, buf0, buf1, buf2, acc_s, sem0, sem1, sem2):
        core = lax.axis_index("c")
        sub = lax.axis_index("s")
        wid = core * NUM_SUBCORES + sub
        stripe = pl.ds(sub * ROWS_PER_TILE, ROWS_PER_TILE)

        @pl.loop(0, ZROWS)
        def _zrow(i):
            @pl.loop(0, d, step=16)
            def _zcol(j):
                buf0[i, pl.ds(j, 16)] = jnp.zeros((16,), jnp.float32)

        @pl.loop(0, ROWS_PER_TILE // ZROWS)
        def _zcopy(k):
            pltpu.sync_copy(
                buf0, acc_s.at[pl.ds(sub * ROWS_PER_TILE + k * ZROWS, ZROWS)])

        pltpu.sync_copy(src_hbm.at[wid], src_v)
        pltpu.sync_copy(dst_hbm.at[wid], dst_v)
        plsc.subcore_barrier()

        cbuf = pl.ds(0, CHUNK)
        bufs = (buf0, buf1, buf2)
        sems = (sem0, sem1, sem2)

        def gather(j, b):
            pltpu.async_copy(sup_hbm.at[src_v.at[j]], bufs[b].at[cbuf], sems[b])

        def consume(j, b):
            pltpu.make_async_copy(
                sup_hbm.at[src_v.at[j]], bufs[b].at[cbuf], sems[b]).wait()
            pltpu.sync_copy(bufs[b].at[cbuf], acc_s.at[dst_v.at[j]], add=True)

        gather(0, 0)
        gather(1, 1)

        @pl.loop(0, (NCHUNKS - 2) // 3)
        def _trip(k):
            j = 3 * k
            gather(j + 2, 2)
            consume(j, 0)
            gather(j + 3, 0)
            consume(j + 1, 1)
            gather(j + 4, 1)
            consume(j + 2, 2)

        consume(NCHUNKS - 2, 0)
        consume(NCHUNKS - 1, 1)

        plsc.subcore_barrier()
        pltpu.sync_copy(acc_s.at[stripe], out_hbm.at[core, sub])

    return spmm


_spmm128 = _make_spmm(D_HIDDEN)
_spmm16 = _make_spmm_narrow(N_CLASSES)


def kernel(x, edge_index, W1, b1, W2, b2):
    src128 = edge_index[0].reshape(NW, NPHASES, PC128, C128)
    dst128 = edge_index[1].reshape(NW, NPHASES, PC128, C128)
    src = edge_index[0].reshape(NW, NCHUNKS, CHUNK)
    dst = edge_index[1].reshape(NW, NCHUNKS, CHUNK)

    support1 = _matmul(x, W1)

    p1 = _spmm128(src128, dst128, support1).reshape(NUM_CORES, N_PAD, D_HIDDEN)

    mask = jax.random.bernoulli(jax.random.key(42), 0.5, (N_NODES, D_HIDDEN))
    mult = jnp.pad(mask.astype(jnp.float32) * 2.0,
                   ((0, N_PAD - N_NODES), (0, 0)))
    w2pad = jnp.zeros((D_HIDDEN, D_HIDDEN), jnp.float32).at[:, :N_CLASSES].set(W2)
    support2 = _fused_mid(p1[0], p1[1], b1.reshape(1, -1), mult, w2pad)
    support2 = support2[:, :N_CLASSES]

    p2 = _spmm16(src, dst, support2).reshape(NUM_CORES, N_PAD, N_CLASSES)

    q0 = p2[0, :N_NODES].reshape(-1, 128)
    q1 = p2[1, :N_NODES].reshape(-1, 128)
    b2t = jnp.tile(b2, 128 // N_CLASSES).reshape(1, 128)
    out = _final_sum(q0, q1, jnp.broadcast_to(b2t, q0.shape))
    return out.reshape(N_NODES, N_CLASSES)
